# baseline (speedup 1.0000x reference)
import functools

import jax
import jax.numpy as jnp
from jax import lax
from jax.experimental import pallas as pl
from jax.experimental.pallas import tpu as pltpu
from jax.experimental.pallas import tpu_sc as plsc

_C = 128
_NBUF = 2


@functools.cache
def _build(H, B, V, D):
    info = plsc.get_sparse_core_info()
    NC, NS, L = info.num_cores, info.num_subcores, info.num_lanes
    NW = NC * NS
    stripe = B // NW
    blocks_per_h = stripe // _C
    n_blocks = H * blocks_per_h
    assert n_blocks % _NBUF == 0
    G = _C // L

    mesh = plsc.VectorSubcoreMesh(core_axis_name="c", subcore_axis_name="s")

    @functools.partial(
        pl.kernel,
        mesh=mesh,
        out_type=jax.ShapeDtypeStruct((H, D, B), jnp.float32),
        scratch_types=[
            pltpu.VMEM((H, stripe), jnp.int32),
            pltpu.VMEM((_NBUF, _C), jnp.int32),
            pltpu.VMEM((_NBUF, _C), jnp.int32),
            pltpu.VMEM((_NBUF, _C, 137), jnp.float32),
            pltpu.VMEM((_NBUF, D, _C), jnp.float32),
            pltpu.SemaphoreType.DMA((_NBUF,)),
            pltpu.SemaphoreType.DMA((_NBUF,)),
        ],
        compiler_params=pltpu.CompilerParams(
            use_tc_tiling_on_sc=True,
            needs_layout_passes=False,
            disable_bounds_checks=True,
        ),
    )
    def gather_kernel(xt_hbm, w2_hbm, out_hbm, idx_all, idxg, vsel, gbuf,
                      out_t, gsems, ssems):
        wid = lax.axis_index("s") * NC + lax.axis_index("c")
        col_base = wid * stripe

        pltpu.sync_copy(xt_hbm.at[:, pl.ds(col_base, stripe)], idx_all)

        def prep_fire(b, s):
            h = s // blocks_per_h
            boff = (s % blocks_per_h) * _C
            for g in range(G):
                v = idx_all[h, pl.ds(boff + g * L, L)]
                idxg[b, pl.ds(g * L, L)] = lax.shift_right_logical(v, 1)
                vsel[b, pl.ds(g * L, L)] = lax.shift_left(
                    lax.bitwise_and(v, 1), 6
                )
            pltpu.async_copy(
                w2_hbm.at[idxg.at[b]],
                gbuf.at[b, :, pl.ds(0, 128)],
                gsems.at[b],
            )

        def wait_gather(b):
            pltpu.make_async_copy(
                w2_hbm.at[pl.ds(0, _C)],
                gbuf.at[b, :, pl.ds(0, 128)],
                gsems.at[b],
            ).wait()

        def transpose(b):
            DU = 8

            def run(g):
                rows = jnp.arange(L, dtype=jnp.int32) + (g * L)
                sel = vsel[b, pl.ds(g * L, L)]

                def dbody(i, carry):
                    d0 = i * DU
                    for dd in range(DU):
                        d = d0 + dd
                        vals = plsc.load_gather(gbuf.at[b], [rows, sel + d])
                        out_t[b, d, pl.ds(g * L, L)] = vals
                    return carry

                lax.fori_loop(0, D // DU, dbody, 0)

            for g in range(G):
                run(g)

        def store_start(b, s):
            h = s // blocks_per_h
            col0 = col_base + (s % blocks_per_h) * _C
            pltpu.async_copy(
                out_t.at[b], out_hbm.at[h, :, pl.ds(col0, _C)], ssems.at[b]
            )

        def wait_store(b):
            pltpu.make_async_copy(
                out_t.at[b], out_hbm.at[0, :, pl.ds(0, _C)], ssems.at[b]
            ).wait()

        prep_fire(0, 0)

        def body(i, carry):
            for b in range(_NBUF):
                s = i * _NBUF + b
                nxt = s + 1
                bf = (b + 1) % _NBUF

                @pl.when(s >= 1)
                def _():
                    wait_store(bf)

                @pl.when(nxt < n_blocks)
                def _():
                    prep_fire(bf, nxt)

                wait_gather(b)
                transpose(b)
                store_start(b, s)
            return carry

        lax.fori_loop(0, n_blocks // _NBUF, body, 0)
        wait_store((n_blocks - 1) % _NBUF)

    return gather_kernel


def kernel(x, weight):
    B, H = x.shape
    V, D = weight.shape
    xt = x.T
    w2 = weight.reshape(V // 2, 2 * D)
    out3 = _build(H, B, V, D)(xt, w2)
    return out3.transpose(2, 0, 1)


# speedup vs baseline: 1.3596x; 1.3596x over previous
import functools

import jax
import jax.numpy as jnp
from jax import lax
from jax.experimental import pallas as pl
from jax.experimental.pallas import tpu as pltpu
from jax.experimental.pallas import tpu_sc as plsc

_C = 128
_NBUF = 2


@functools.cache
def _build(H, B, V, D):
    info = plsc.get_sparse_core_info()
    NC, NS, L = info.num_cores, info.num_subcores, info.num_lanes
    NW = NC * NS
    stripe = B // NW
    blocks_per_h = stripe // _C
    n_blocks = H * blocks_per_h
    assert n_blocks % _NBUF == 0
    G = _C // L

    mesh = plsc.VectorSubcoreMesh(core_axis_name="c", subcore_axis_name="s")

    @functools.partial(
        pl.kernel,
        mesh=mesh,
        out_type=jax.ShapeDtypeStruct((H, D, B), jnp.float32),
        scratch_types=[
            pltpu.VMEM((H, stripe), jnp.int32),
            pltpu.VMEM((_NBUF, _C), jnp.int32),
            pltpu.VMEM((_NBUF, _C), jnp.int32),
            pltpu.VMEM((_NBUF, _C, 137), jnp.float32),
            pltpu.VMEM((_NBUF, D, _C), jnp.float32),
            pltpu.SemaphoreType.DMA((_NBUF,)),
            pltpu.SemaphoreType.DMA((_NBUF,)),
        ],
        compiler_params=pltpu.CompilerParams(
            use_tc_tiling_on_sc=True,
            needs_layout_passes=False,
            disable_bounds_checks=True,
        ),
    )
    def gather_kernel(xt_hbm, w2_hbm, out_hbm, idx_all, idxg, vsel, gbuf,
                      out_t, gsems, ssems):
        wid = lax.axis_index("s") * NC + lax.axis_index("c")
        col_base = wid * stripe

        pltpu.sync_copy(xt_hbm.at[:, pl.ds(col_base, stripe)], idx_all)

        def prep_fire(b, s):
            h = s // blocks_per_h
            boff = (s % blocks_per_h) * _C
            for g in range(G):
                v = idx_all[h, pl.ds(boff + g * L, L)]
                idxg[b, pl.ds(g * L, L)] = lax.shift_right_logical(v, 1)
                vsel[b, pl.ds(g * L, L)] = lax.shift_left(
                    lax.bitwise_and(v, 1), 6
                )
            pltpu.async_copy(
                w2_hbm.at[idxg.at[b]],
                gbuf.at[b, :, pl.ds(0, 128)],
                gsems.at[b],
            )

        def wait_gather(b):
            pltpu.make_async_copy(
                w2_hbm.at[pl.ds(0, _C)],
                gbuf.at[b, :, pl.ds(0, 128)],
                gsems.at[b],
            ).wait()

        def transpose(b):
            def run(g):
                rows = jnp.arange(L, dtype=jnp.int32) + (g * L)
                sel = vsel[b, pl.ds(g * L, L)]

                @plsc.parallel_loop(0, D, 1, unroll=8)
                def _(d):
                    vals = plsc.load_gather(gbuf.at[b], [rows, sel + d])
                    out_t[b, d, pl.ds(g * L, L)] = vals

            for g in range(G):
                run(g)

        def store_start(b, s):
            h = s // blocks_per_h
            col0 = col_base + (s % blocks_per_h) * _C
            pltpu.async_copy(
                out_t.at[b], out_hbm.at[h, :, pl.ds(col0, _C)], ssems.at[b]
            )

        def wait_store(b):
            pltpu.make_async_copy(
                out_t.at[b], out_hbm.at[0, :, pl.ds(0, _C)], ssems.at[b]
            ).wait()

        prep_fire(0, 0)

        def body(i, carry):
            for b in range(_NBUF):
                s = i * _NBUF + b
                nxt = s + 1
                bf = (b + 1) % _NBUF

                @pl.when(s >= 1)
                def _():
                    wait_store(bf)

                @pl.when(nxt < n_blocks)
                def _():
                    prep_fire(bf, nxt)

                wait_gather(b)
                transpose(b)
                store_start(b, s)
            return carry

        lax.fori_loop(0, n_blocks // _NBUF, body, 0)
        wait_store((n_blocks - 1) % _NBUF)

    return gather_kernel


def kernel(x, weight):
    B, H = x.shape
    V, D = weight.shape
    xt = x.T
    w2 = weight.reshape(V // 2, 2 * D)
    out3 = _build(H, B, V, D)(xt, w2)
    return out3.transpose(2, 0, 1)


# speedup vs baseline: 1.4107x; 1.0376x over previous
import functools

import jax
import jax.numpy as jnp
from jax import lax
from jax.experimental import pallas as pl
from jax.experimental.pallas import tpu as pltpu
from jax.experimental.pallas import tpu_sc as plsc

_C = 128
_NBUF = 2


@functools.cache
def _build(H, B, V, D):
    info = plsc.get_sparse_core_info()
    NC, NS, L = info.num_cores, info.num_subcores, info.num_lanes
    NW = NC * NS
    stripe = B // NW
    blocks_per_h = stripe // _C
    n_blocks = H * blocks_per_h
    assert n_blocks % _NBUF == 0
    G = _C // L

    mesh = plsc.VectorSubcoreMesh(core_axis_name="c", subcore_axis_name="s")

    @functools.partial(
        pl.kernel,
        mesh=mesh,
        out_type=jax.ShapeDtypeStruct((H, D, B), jnp.float32),
        scratch_types=[
            pltpu.VMEM((H, stripe), jnp.int32),
            pltpu.VMEM((_NBUF, _C), jnp.int32),
            pltpu.VMEM((_NBUF, _C), jnp.int32),
            pltpu.VMEM((_NBUF, _C, 137), jnp.float32),
            pltpu.VMEM((_NBUF, D, _C), jnp.float32),
            pltpu.SemaphoreType.DMA((_NBUF,)),
            pltpu.SemaphoreType.DMA((_NBUF,)),
        ],
        compiler_params=pltpu.CompilerParams(
            use_tc_tiling_on_sc=True,
            needs_layout_passes=False,
            disable_bounds_checks=True,
        ),
    )
    def gather_kernel(xt_hbm, w2_hbm, out_hbm, idx_all, idxg, vsel, gbuf,
                      out_t, gsems, ssems):
        wid = lax.axis_index("s") * NC + lax.axis_index("c")
        col_base = wid * stripe

        pltpu.sync_copy(xt_hbm.at[:, pl.ds(col_base, stripe)], idx_all)

        def prep_fire(b, s):
            h = s // blocks_per_h
            boff = (s % blocks_per_h) * _C
            for g in range(G):
                v = idx_all[h, pl.ds(boff + g * L, L)]
                idxg[b, pl.ds(g * L, L)] = lax.shift_right_logical(v, 1)
                vsel[b, pl.ds(g * L, L)] = lax.shift_left(
                    lax.bitwise_and(v, 1), 6
                )
            pltpu.async_copy(
                w2_hbm.at[idxg.at[b]],
                gbuf.at[b, :, pl.ds(0, 128)],
                gsems.at[b],
            )

        def wait_gather(b):
            pltpu.make_async_copy(
                w2_hbm.at[pl.ds(0, _C)],
                gbuf.at[b, :, pl.ds(0, 128)],
                gsems.at[b],
            ).wait()

        def transpose(b):
            rows_l = [
                jnp.arange(L, dtype=jnp.int32) + (g * L) for g in range(G)
            ]
            sel_l = [vsel[b, pl.ds(g * L, L)] for g in range(G)]

            @plsc.parallel_loop(0, D, 1, unroll=4)
            def _(d):
                for g in range(G):
                    vals = plsc.load_gather(
                        gbuf.at[b], [rows_l[g], sel_l[g] + d]
                    )
                    out_t[b, d, pl.ds(g * L, L)] = vals

        def store_start(b, s):
            h = s // blocks_per_h
            col0 = col_base + (s % blocks_per_h) * _C
            pltpu.async_copy(
                out_t.at[b], out_hbm.at[h, :, pl.ds(col0, _C)], ssems.at[b]
            )

        def wait_store(b):
            pltpu.make_async_copy(
                out_t.at[b], out_hbm.at[0, :, pl.ds(0, _C)], ssems.at[b]
            ).wait()

        prep_fire(0, 0)

        def body(i, carry):
            for b in range(_NBUF):
                s = i * _NBUF + b
                nxt = s + 1
                bf = (b + 1) % _NBUF

                @pl.when(s >= 1)
                def _():
                    wait_store(bf)

                @pl.when(nxt < n_blocks)
                def _():
                    prep_fire(bf, nxt)

                wait_gather(b)
                transpose(b)
                store_start(b, s)
            return carry

        lax.fori_loop(0, n_blocks // _NBUF, body, 0)
        wait_store((n_blocks - 1) % _NBUF)

    return gather_kernel


def kernel(x, weight):
    B, H = x.shape
    V, D = weight.shape
    xt = x.T
    w2 = weight.reshape(V // 2, 2 * D)
    out3 = _build(H, B, V, D)(xt, w2)
    return out3.transpose(2, 0, 1)
